# baseline (speedup 1.0000x reference)
import jax
import jax.numpy as jnp
from jax import lax
from jax.experimental import pallas as pl
from jax.experimental.pallas import tpu as pltpu
from jax.experimental.pallas import tpu_sc as plsc

N_NODES = 10000
N_PAD = 10240
D = 128
HEADS = 8
HD = 16
N_EDGES = 320000

NC = 2
NS = 16
NW = NC * NS
R = N_PAD // NW
ACCW = 144
ECHUNK = 2000
N_ECHUNKS = N_EDGES // ECHUNK
PKCAP = 12032
PADPK = (R - 1) << 14

_HIGH = jax.lax.Precision.HIGHEST


def _pre_body(x_ref, w_ref, am_ref, ad_ref, h_ref, as_ref, adr_ref):
    h = jax.lax.dot(x_ref[...], w_ref[...], precision=_HIGH)
    h_ref[...] = h
    as_ref[...] = jax.lax.dot(h, am_ref[...], precision=_HIGH)
    adr_ref[...] = jax.lax.dot(h, ad_ref[...], precision=_HIGH)


def _dense_pre(x, W, AsM, AdM):
    blk = 1000
    grid = N_NODES // blk
    return pl.pallas_call(
        _pre_body,
        grid=(grid,),
        in_specs=[
            pl.BlockSpec((blk, D), lambda i: (i, 0)),
            pl.BlockSpec((D, D), lambda i: (0, 0)),
            pl.BlockSpec((D, 16), lambda i: (0, 0)),
            pl.BlockSpec((D, 16), lambda i: (0, 0)),
        ],
        out_specs=[
            pl.BlockSpec((blk, D), lambda i: (i, 0)),
            pl.BlockSpec((blk, 16), lambda i: (i, 0)),
            pl.BlockSpec((blk, 16), lambda i: (i, 0)),
        ],
        out_shape=[
            jax.ShapeDtypeStruct((N_NODES, D), jnp.float32),
            jax.ShapeDtypeStruct((N_NODES, 16), jnp.float32),
            jax.ShapeDtypeStruct((N_NODES, 16), jnp.float32),
        ],
    )(x, W, AsM, AdM)


def _edge_body(hb_hbm, asb_hbm, adp_hbm, pk_hbm,
               pm_hbm,
               acc, pkbuf, asbv, advl,
               ebf0, ebf1, hv0, hv1,
               sidx0, sidx1, didx0, didx1,
               st0, st1, sg0, sg1):
    ebf = (ebf0, ebf1)
    hv = (hv0, hv1)
    sidx = (sidx0, sidx1)
    didx = (didx0, didx1)
    st = (st0, st1)
    sg = (sg0, sg1)

    c = lax.axis_index("c")
    s = lax.axis_index("s")
    w = c * NS + s

    zeros16 = jnp.zeros((16,), jnp.float32)
    iota16 = lax.iota(jnp.int32, 16)
    _ilv = plsc.PackFormat.INTERLEAVED

    def zfill(i, carry):
        for k in range(ACCW // 16):
            acc[i, pl.ds(k * 16, 16)] = zeros16
        return carry

    lax.fori_loop(0, R, zfill, 0)

    pltpu.sync_copy(asb_hbm, asbv)
    pltpu.sync_copy(adp_hbm.at[w], advl)

    def stage(ci, b):
        pltpu.async_copy(pk_hbm.at[pl.ds(ci * ECHUNK, ECHUNK)], ebf[b],
                         st[b])

    def wait_stage(ci, b):
        pltpu.make_async_copy(pk_hbm.at[pl.ds(ci * ECHUNK, ECHUNK)],
                              ebf[b], st[b]).wait()

    stage(0, 0)
    stage(1, 1)

    def scan_pair(i, o):
        for b in (0, 1):
            ci = 2 * i + b
            wait_stage(ci, b)

            def scan16(j, oo):
                pkin = ebf[b][pl.ds(j * 16, 16)]
                d16 = pkin >> 14
                m = (d16 & 31) == w
                entry = (pkin & 16383) | ((pkin >> 19) << 14)
                plsc.store_compressed(pkbuf.at[pl.ds(oo, 16)], entry,
                                      mask=m)
                cnt = plsc.all_reduce_population_count(m)
                return oo + cnt[0]

            o = lax.fori_loop(0, ECHUNK // 16, scan16, o)

            @pl.when(ci + 2 < N_ECHUNKS)
            def _():
                stage(ci + 2, b)
        return o

    o = lax.fori_loop(0, N_ECHUNKS // 2, scan_pair, jnp.int32(0))

    padvec = jnp.full((16,), PADPK, jnp.int32)
    pkbuf[pl.ds(o, 16)] = padvec
    pkbuf[pl.ds(o + 16, 16)] = padvec
    ng = ((o + 31) // 32) * 2

    def issue(g, b):
        pk = pkbuf[pl.ds(g * 16, 16)]
        sidx[b][...] = pk & 16383
        didx[b][...] = pk >> 14
        pltpu.async_copy(hb_hbm.at[sidx[b]], hv[b], sg[b])

    def wait_group(b):
        pltpu.make_async_copy(hb_hbm.at[sidx[b]], hv[b], sg[b]).wait()

    def accumulate(b):
        d16 = didx[b][...]
        s16v = sidx[b][...]
        for p in range(HEADS // 2):
            was = plsc.load_gather(asbv, [s16v * 4 + p])
            a_lo, a_hi = plsc.unpack(plsc.bitcast(was, jnp.bfloat16),
                                     format=_ilv)
            for hh, av in ((2 * p, a_lo), (2 * p + 1, a_hi)):
                dv = plsc.load_gather(advl, [d16,
                                             jnp.full((16,), hh, jnp.int32)])
                e = av + dv
                e = jnp.maximum(e, 0.2 * e)
                sv = jnp.exp(e)
                plsc.addupdate_scatter(acc, [d16, jnp.full((16,), D + hh,
                                                           jnp.int32)], sv)
                for cw in range(HD // 2):
                    wv = plsc.load_gather(
                        hv[b], [iota16,
                                jnp.full((16,), hh * 8 + cw, jnp.int32)])
                    c_lo, c_hi = plsc.unpack(plsc.bitcast(wv, jnp.bfloat16),
                                             format=_ilv)
                    base = hh * HD + 2 * cw
                    plsc.addupdate_scatter(
                        acc, [d16, jnp.full((16,), base, jnp.int32)],
                        c_lo * sv)
                    plsc.addupdate_scatter(
                        acc, [d16, jnp.full((16,), base + 1, jnp.int32)],
                        c_hi * sv)

    issue(0, 0)
    issue(1, 1)

    def group_pair(i, carry):
        for b in (0, 1):
            g = 2 * i + b
            wait_group(b)
            accumulate(b)

            @pl.when(g + 2 < ng)
            def _():
                issue(g + 2, b)
        return carry

    lax.fori_loop(0, ng // 2, group_pair, 0)

    pltpu.sync_copy(acc, pm_hbm.at[w])


def _edge_call(hb_pk, asb_pk, adP, pkin):
    mesh = plsc.VectorSubcoreMesh(core_axis_name="c", subcore_axis_name="s")
    fn = pl.kernel(
        _edge_body,
        out_type=jax.ShapeDtypeStruct((NW, R, ACCW), jnp.float32),
        mesh=mesh,
        scratch_types=[
            pltpu.VMEM((R, ACCW), jnp.float32),
            pltpu.VMEM((PKCAP,), jnp.int32),
            pltpu.VMEM((N_PAD * HEADS // 2,), jnp.int32),
            pltpu.VMEM((R, 16), jnp.float32),
            pltpu.VMEM((ECHUNK,), jnp.int32),
            pltpu.VMEM((ECHUNK,), jnp.int32),
            pltpu.VMEM((16, D // 2), jnp.int32),
            pltpu.VMEM((16, D // 2), jnp.int32),
            pltpu.VMEM((16,), jnp.int32),
            pltpu.VMEM((16,), jnp.int32),
            pltpu.VMEM((16,), jnp.int32),
            pltpu.VMEM((16,), jnp.int32),
            pltpu.SemaphoreType.DMA,
            pltpu.SemaphoreType.DMA,
            pltpu.SemaphoreType.DMA,
            pltpu.SemaphoreType.DMA,
        ],
        compiler_params=pltpu.CompilerParams(
            needs_layout_passes=False, use_tc_tiling_on_sc=False),
    )
    return fn(hb_pk, asb_pk, adP, pkin)


def _comb_body(pm_ref, h_ref, as_ref, ad_ref, b_ref, o_ref):
    pmb = pm_ref[0]
    msg = pmb[:, :D]
    den8 = pmb[:, D:D + HEADS]
    e = as_ref[0][:, :HEADS] + ad_ref[0][:, :HEADS]
    e = jnp.maximum(e, 0.2 * e)
    sself = jnp.exp(e)
    den = den8 + sself
    rows = lax.broadcasted_iota(jnp.int32, (HEADS, D), 0)
    cols = lax.broadcasted_iota(jnp.int32, (HEADS, D), 1)
    expand = (cols // HD == rows).astype(jnp.float32)
    den128 = jax.lax.dot(den, expand, precision=_HIGH)
    s128 = jax.lax.dot(sself, expand, precision=_HIGH)
    out = (msg + h_ref[0] * s128) / den128 + b_ref[...]
    o_ref[0] = jnp.maximum(out, 0.0)


def _combine(pm, hP, asP, adP, bias2d):
    return pl.pallas_call(
        _comb_body,
        grid=(NW,),
        in_specs=[
            pl.BlockSpec((1, R, ACCW), lambda t: (t, 0, 0)),
            pl.BlockSpec((1, R, D), lambda t: (t, 0, 0)),
            pl.BlockSpec((1, R, 16), lambda t: (t, 0, 0)),
            pl.BlockSpec((1, R, 16), lambda t: (t, 0, 0)),
            pl.BlockSpec((1, D), lambda t: (0, 0)),
        ],
        out_specs=pl.BlockSpec((1, R, D), lambda t: (t, 0, 0)),
        out_shape=jax.ShapeDtypeStruct((NW, R, D), jnp.float32),
    )(pm, hP, asP, adP, bias2d)


def kernel(x, edge_index, W, att_src, att_dst, bias):
    src_lin = edge_index[0].astype(jnp.int32)
    dst_lin = edge_index[1].astype(jnp.int32)

    eye = jnp.eye(HEADS, dtype=jnp.float32)
    a_s = att_src.reshape(HEADS, HD)
    a_d = att_dst.reshape(HEADS, HD)
    AsM = (a_s[:, :, None] * eye[:, None, :]).reshape(D, HEADS)
    AdM = (a_d[:, :, None] * eye[:, None, :]).reshape(D, HEADS)
    pad = jnp.zeros((D, 16 - HEADS), jnp.float32)
    AsM = jnp.concatenate([AsM, pad], axis=1)
    AdM = jnp.concatenate([AdM, pad], axis=1)

    h, as16, ad16 = _dense_pre(x, W, AsM, AdM)

    hpad = jnp.pad(h, ((0, N_PAD - N_NODES), (0, 0)))
    aspad = jnp.pad(as16, ((0, N_PAD - N_NODES), (0, 0)))
    adpad = jnp.pad(ad16, ((0, N_PAD - N_NODES), (0, 0)))
    hP = hpad.reshape(R, NW, D).transpose(1, 0, 2)
    asP = aspad.reshape(R, NW, 16).transpose(1, 0, 2)
    adP = adpad.reshape(R, NW, 16).transpose(1, 0, 2)

    hb_pk = lax.bitcast_convert_type(
        hpad.astype(jnp.bfloat16).reshape(N_PAD, D // 2, 2), jnp.int32)
    asb_pk = lax.bitcast_convert_type(
        aspad[:, :HEADS].astype(jnp.bfloat16).reshape(N_PAD, HEADS // 2, 2),
        jnp.int32).reshape(N_PAD * HEADS // 2)
    pkin = src_lin | (dst_lin << 14)

    pm = _edge_call(hb_pk, asb_pk, adP, pkin)

    bias2d = bias.reshape(1, D)
    outP = _combine(pm, hP, asP, adP, bias2d)
    return outP.transpose(1, 0, 2).reshape(N_PAD, D)[:N_NODES]


# speedup vs baseline: 1.3497x; 1.3497x over previous
import jax
import jax.numpy as jnp
from jax import lax
from jax.experimental import pallas as pl
from jax.experimental.pallas import tpu as pltpu
from jax.experimental.pallas import tpu_sc as plsc

N_NODES = 10000
N_PAD = 10240
D = 128
HEADS = 8
HD = 16
N_EDGES = 320000

NC = 2
NS = 16
NW = NC * NS
R = N_PAD // NW
ACCW = 144
ECHUNK = 2000
N_ECHUNKS = N_EDGES // ECHUNK
PKCAP = 12032
PADPK = (R - 1) << 14

_HIGH = jax.lax.Precision.HIGHEST


def _pre_body(x_ref, w_ref, am_ref, ad_ref, h_ref, as_ref, adr_ref):
    h = jax.lax.dot(x_ref[...], w_ref[...], precision=_HIGH)
    h_ref[...] = h
    as_ref[...] = jax.lax.dot(h, am_ref[...], precision=_HIGH)
    adr_ref[...] = jax.lax.dot(h, ad_ref[...], precision=_HIGH)


def _dense_pre(x, W, AsM, AdM):
    blk = 1000
    grid = N_NODES // blk
    return pl.pallas_call(
        _pre_body,
        grid=(grid,),
        in_specs=[
            pl.BlockSpec((blk, D), lambda i: (i, 0)),
            pl.BlockSpec((D, D), lambda i: (0, 0)),
            pl.BlockSpec((D, 16), lambda i: (0, 0)),
            pl.BlockSpec((D, 16), lambda i: (0, 0)),
        ],
        out_specs=[
            pl.BlockSpec((blk, D), lambda i: (i, 0)),
            pl.BlockSpec((blk, 16), lambda i: (i, 0)),
            pl.BlockSpec((blk, 16), lambda i: (i, 0)),
        ],
        out_shape=[
            jax.ShapeDtypeStruct((N_NODES, D), jnp.float32),
            jax.ShapeDtypeStruct((N_NODES, 16), jnp.float32),
            jax.ShapeDtypeStruct((N_NODES, 16), jnp.float32),
        ],
    )(x, W, AsM, AdM)


def _edge_body(hb_hbm, asb_hbm, adp_hbm, pk_hbm,
               pm_hbm,
               acc, pkbuf, asbv, advl,
               ebf0, ebf1, hv0, hv1,
               sidx0, sidx1, didx0, didx1,
               st0, st1, sg0, sg1):
    ebf = (ebf0, ebf1)
    hv = (hv0, hv1)
    sidx = (sidx0, sidx1)
    didx = (didx0, didx1)
    st = (st0, st1)
    sg = (sg0, sg1)

    c = lax.axis_index("c")
    s = lax.axis_index("s")
    w = c * NS + s

    zeros16 = jnp.zeros((16,), jnp.float32)
    iota16 = lax.iota(jnp.int32, 16)
    _ilv = plsc.PackFormat.INTERLEAVED

    def zfill(i, carry):
        for k in range(R // 16):
            acc[i, pl.ds(k * 16, 16)] = zeros16
        return carry

    lax.fori_loop(0, ACCW, zfill, 0)

    pltpu.sync_copy(asb_hbm, asbv)
    pltpu.sync_copy(adp_hbm.at[w], advl)

    def stage(ci, b):
        pltpu.async_copy(pk_hbm.at[pl.ds(ci * ECHUNK, ECHUNK)], ebf[b],
                         st[b])

    def wait_stage(ci, b):
        pltpu.make_async_copy(pk_hbm.at[pl.ds(ci * ECHUNK, ECHUNK)],
                              ebf[b], st[b]).wait()

    stage(0, 0)
    stage(1, 1)

    def scan_pair(i, o):
        for b in (0, 1):
            ci = 2 * i + b
            wait_stage(ci, b)

            def scan16(j, oo):
                pkin = ebf[b][pl.ds(j * 16, 16)]
                d16 = pkin >> 14
                m = (d16 & 31) == w
                entry = (pkin & 16383) | ((pkin >> 19) << 14)
                plsc.store_compressed(pkbuf.at[pl.ds(oo, 16)], entry,
                                      mask=m)
                cnt = plsc.all_reduce_population_count(m)
                return oo + cnt[0]

            o = lax.fori_loop(0, ECHUNK // 16, scan16, o)

            @pl.when(ci + 2 < N_ECHUNKS)
            def _():
                stage(ci + 2, b)
        return o

    o = lax.fori_loop(0, N_ECHUNKS // 2, scan_pair, jnp.int32(0))

    padvec = jnp.full((16,), PADPK, jnp.int32)
    pkbuf[pl.ds(o, 16)] = padvec
    pkbuf[pl.ds(o + 16, 16)] = padvec
    ng = ((o + 31) // 32) * 2

    def issue(g, b):
        pk = pkbuf[pl.ds(g * 16, 16)]
        sidx[b][...] = pk & 16383
        didx[b][...] = pk >> 14
        pltpu.async_copy(hb_hbm.at[sidx[b]], hv[b], sg[b])

    def wait_group(b):
        pltpu.make_async_copy(hb_hbm.at[sidx[b]], hv[b], sg[b]).wait()

    def accumulate(b):
        d16 = didx[b][...]
        s16v = sidx[b][...]
        for p in range(HEADS // 2):
            was = plsc.load_gather(asbv, [s16v * 4 + p])
            a_lo, a_hi = plsc.unpack(plsc.bitcast(was, jnp.bfloat16),
                                     format=_ilv)
            for hh, av in ((2 * p, a_lo), (2 * p + 1, a_hi)):
                dv = plsc.load_gather(advl, [jnp.full((16,), hh, jnp.int32),
                                             d16])
                e = av + dv
                e = jnp.maximum(e, 0.2 * e)
                sv = jnp.exp(e)
                plsc.addupdate_scatter(acc, [jnp.full((16,), D + hh,
                                                      jnp.int32), d16], sv)
                for cw in range(HD // 2):
                    wv = plsc.load_gather(
                        hv[b], [iota16,
                                jnp.full((16,), hh * 8 + cw, jnp.int32)])
                    c_lo, c_hi = plsc.unpack(plsc.bitcast(wv, jnp.bfloat16),
                                             format=_ilv)
                    base = hh * HD + 2 * cw
                    plsc.addupdate_scatter(
                        acc, [jnp.full((16,), base, jnp.int32), d16],
                        c_lo * sv)
                    plsc.addupdate_scatter(
                        acc, [jnp.full((16,), base + 1, jnp.int32), d16],
                        c_hi * sv)

    issue(0, 0)
    issue(1, 1)

    def group_pair(i, carry):
        for b in (0, 1):
            g = 2 * i + b
            wait_group(b)
            accumulate(b)

            @pl.when(g + 2 < ng)
            def _():
                issue(g + 2, b)
        return carry

    lax.fori_loop(0, ng // 2, group_pair, 0)

    pltpu.sync_copy(acc, pm_hbm.at[w])


def _edge_call(hb_pk, asb_pk, adP, pkin):
    mesh = plsc.VectorSubcoreMesh(core_axis_name="c", subcore_axis_name="s")
    fn = pl.kernel(
        _edge_body,
        out_type=jax.ShapeDtypeStruct((NW, ACCW, R), jnp.float32),
        mesh=mesh,
        scratch_types=[
            pltpu.VMEM((ACCW, R), jnp.float32),
            pltpu.VMEM((PKCAP,), jnp.int32),
            pltpu.VMEM((N_PAD * HEADS // 2,), jnp.int32),
            pltpu.VMEM((16, R), jnp.float32),
            pltpu.VMEM((ECHUNK,), jnp.int32),
            pltpu.VMEM((ECHUNK,), jnp.int32),
            pltpu.VMEM((16, D // 2), jnp.int32),
            pltpu.VMEM((16, D // 2), jnp.int32),
            pltpu.VMEM((16,), jnp.int32),
            pltpu.VMEM((16,), jnp.int32),
            pltpu.VMEM((16,), jnp.int32),
            pltpu.VMEM((16,), jnp.int32),
            pltpu.SemaphoreType.DMA,
            pltpu.SemaphoreType.DMA,
            pltpu.SemaphoreType.DMA,
            pltpu.SemaphoreType.DMA,
        ],
        compiler_params=pltpu.CompilerParams(
            needs_layout_passes=False, use_tc_tiling_on_sc=False),
    )
    return fn(hb_pk, asb_pk, adP, pkin)


def _comb_body(pm_ref, h_ref, as_ref, ad_ref, b_ref, o_ref):
    pmb = pm_ref[0]
    msg = pmb[:D, :]
    den8 = pmb[D:D + HEADS, :]
    e = as_ref[0][:HEADS, :] + ad_ref[0][:HEADS, :]
    e = jnp.maximum(e, 0.2 * e)
    sself = jnp.exp(e)
    den = den8 + sself
    rows = lax.broadcasted_iota(jnp.int32, (D, HEADS), 0)
    cols = lax.broadcasted_iota(jnp.int32, (D, HEADS), 1)
    expand = (rows // HD == cols).astype(jnp.float32)
    den128 = jax.lax.dot(expand, den, precision=_HIGH)
    s128 = jax.lax.dot(expand, sself, precision=_HIGH)
    out = (msg + h_ref[0] * s128) / den128 + b_ref[...]
    o_ref[0] = jnp.maximum(out, 0.0)


def _combine(pm, hPT, asPT, adPT, biasT):
    return pl.pallas_call(
        _comb_body,
        grid=(NW,),
        in_specs=[
            pl.BlockSpec((1, ACCW, R), lambda t: (t, 0, 0)),
            pl.BlockSpec((1, D, R), lambda t: (t, 0, 0)),
            pl.BlockSpec((1, 16, R), lambda t: (t, 0, 0)),
            pl.BlockSpec((1, 16, R), lambda t: (t, 0, 0)),
            pl.BlockSpec((D, 1), lambda t: (0, 0)),
        ],
        out_specs=pl.BlockSpec((1, D, R), lambda t: (t, 0, 0)),
        out_shape=jax.ShapeDtypeStruct((NW, D, R), jnp.float32),
    )(pm, hPT, asPT, adPT, biasT)


def kernel(x, edge_index, W, att_src, att_dst, bias):
    src_lin = edge_index[0].astype(jnp.int32)
    dst_lin = edge_index[1].astype(jnp.int32)

    eye = jnp.eye(HEADS, dtype=jnp.float32)
    a_s = att_src.reshape(HEADS, HD)
    a_d = att_dst.reshape(HEADS, HD)
    AsM = (a_s[:, :, None] * eye[:, None, :]).reshape(D, HEADS)
    AdM = (a_d[:, :, None] * eye[:, None, :]).reshape(D, HEADS)
    pad = jnp.zeros((D, 16 - HEADS), jnp.float32)
    AsM = jnp.concatenate([AsM, pad], axis=1)
    AdM = jnp.concatenate([AdM, pad], axis=1)

    h, as16, ad16 = _dense_pre(x, W, AsM, AdM)

    hpad = jnp.pad(h, ((0, N_PAD - N_NODES), (0, 0)))
    aspad = jnp.pad(as16, ((0, N_PAD - N_NODES), (0, 0)))
    adpad = jnp.pad(ad16, ((0, N_PAD - N_NODES), (0, 0)))
    hPT = hpad.reshape(R, NW, D).transpose(1, 2, 0)
    asPT = aspad.reshape(R, NW, 16).transpose(1, 2, 0)
    adPT = adpad.reshape(R, NW, 16).transpose(1, 2, 0)

    hb_pk = lax.bitcast_convert_type(
        hpad.astype(jnp.bfloat16).reshape(N_PAD, D // 2, 2), jnp.int32)
    asb_pk = lax.bitcast_convert_type(
        aspad[:, :HEADS].astype(jnp.bfloat16).reshape(N_PAD, HEADS // 2, 2),
        jnp.int32).reshape(N_PAD * HEADS // 2)
    pkin = src_lin | (dst_lin << 14)

    pm = _edge_call(hb_pk, asb_pk, adPT, pkin)

    biasT = bias.reshape(D, 1)
    outPT = _combine(pm, hPT, asPT, adPT, biasT)
    return outPT.transpose(2, 0, 1).reshape(N_PAD, D)[:N_NODES]


# speedup vs baseline: 1.7738x; 1.3143x over previous
import jax
import jax.numpy as jnp
from jax import lax
from jax.experimental import pallas as pl
from jax.experimental.pallas import tpu as pltpu
from jax.experimental.pallas import tpu_sc as plsc

N_NODES = 10000
N_PAD = 10240
D = 128
HEADS = 8
HD = 16
N_EDGES = 320000

NC = 2
NS = 16
NW = NC * NS
R = N_PAD // NW
ACCW = 144
ECHUNK = 2000
N_ECHUNKS = N_EDGES // ECHUNK
PKCAP = 12032
PADPK = (R - 1) << 14

_HIGH = jax.lax.Precision.HIGHEST


def _pre_body(x_ref, w_ref, am_ref, ad_ref, h_ref, as_ref, adr_ref):
    h = jax.lax.dot(x_ref[...], w_ref[...], precision=_HIGH)
    h_ref[...] = h
    as_ref[...] = jax.lax.dot(h, am_ref[...], precision=_HIGH)
    adr_ref[...] = jax.lax.dot(h, ad_ref[...], precision=_HIGH)


def _dense_pre(x, W, AsM, AdM):
    blk = 1000
    grid = N_NODES // blk
    return pl.pallas_call(
        _pre_body,
        grid=(grid,),
        in_specs=[
            pl.BlockSpec((blk, D), lambda i: (i, 0)),
            pl.BlockSpec((D, D), lambda i: (0, 0)),
            pl.BlockSpec((D, 16), lambda i: (0, 0)),
            pl.BlockSpec((D, 16), lambda i: (0, 0)),
        ],
        out_specs=[
            pl.BlockSpec((blk, D), lambda i: (i, 0)),
            pl.BlockSpec((blk, 16), lambda i: (i, 0)),
            pl.BlockSpec((blk, 16), lambda i: (i, 0)),
        ],
        out_shape=[
            jax.ShapeDtypeStruct((N_NODES, D), jnp.float32),
            jax.ShapeDtypeStruct((N_NODES, 16), jnp.float32),
            jax.ShapeDtypeStruct((N_NODES, 16), jnp.float32),
        ],
    )(x, W, AsM, AdM)


def _edge_body(hb_hbm, asb_hbm, adp_hbm, pk_hbm,
               pm_hbm,
               acc, pkbuf, asbv, advl,
               ebf0, ebf1, hv0, hv1,
               sidx0, sidx1, didx0, didx1,
               st0, st1, sg0, sg1):
    ebf = (ebf0, ebf1)
    hv = (hv0, hv1)
    sidx = (sidx0, sidx1)
    didx = (didx0, didx1)
    st = (st0, st1)
    sg = (sg0, sg1)

    c = lax.axis_index("c")
    s = lax.axis_index("s")
    w = c * NS + s

    zeros16 = jnp.zeros((16,), jnp.float32)
    iota16 = lax.iota(jnp.int32, 16)
    _ilv = plsc.PackFormat.INTERLEAVED

    def zfill(i, carry):
        for k in range(R // 16):
            acc[i, pl.ds(k * 16, 16)] = zeros16
        return carry

    lax.fori_loop(0, ACCW, zfill, 0)

    pltpu.sync_copy(asb_hbm, asbv)
    pltpu.sync_copy(adp_hbm.at[w], advl)

    def stage(ci, b):
        pltpu.async_copy(pk_hbm.at[pl.ds(ci * ECHUNK, ECHUNK)], ebf[b],
                         st[b])

    def wait_stage(ci, b):
        pltpu.make_async_copy(pk_hbm.at[pl.ds(ci * ECHUNK, ECHUNK)],
                              ebf[b], st[b]).wait()

    stage(0, 0)
    stage(1, 1)

    def scan_pair(i, o):
        for b in (0, 1):
            ci = 2 * i + b
            wait_stage(ci, b)

            def scan16(j, oo):
                pkin = ebf[b][pl.ds(j * 16, 16)]
                d16 = pkin >> 14
                m = (d16 & 31) == w
                entry = (pkin & 16383) | ((pkin >> 19) << 14)
                plsc.store_compressed(pkbuf.at[pl.ds(oo, 16)], entry,
                                      mask=m)
                cnt = plsc.all_reduce_population_count(m)
                return oo + cnt[0]

            o = lax.fori_loop(0, ECHUNK // 16, scan16, o)

            @pl.when(ci + 2 < N_ECHUNKS)
            def _():
                stage(ci + 2, b)
        return o

    o = lax.fori_loop(0, N_ECHUNKS // 2, scan_pair, jnp.int32(0))

    padvec = jnp.full((16,), PADPK, jnp.int32)
    pkbuf[pl.ds(o, 16)] = padvec
    pkbuf[pl.ds(o + 16, 16)] = padvec
    ng = ((o + 31) // 32) * 2

    def issue(g, b):
        pk = pkbuf[pl.ds(g * 16, 16)]
        sidx[b][...] = pk & 16383
        didx[b][...] = pk >> 14
        pltpu.async_copy(hb_hbm.at[sidx[b]], hv[b], sg[b])

    def wait_group(b):
        pltpu.make_async_copy(hb_hbm.at[sidx[b]], hv[b], sg[b]).wait()

    def accumulate(b):
        d16 = didx[b][...]
        s16v = sidx[b][...]
        for p in range(HEADS // 2):
            was = plsc.load_gather(asbv, [s16v * 4 + p])
            a_lo, a_hi = plsc.unpack(plsc.bitcast(was, jnp.bfloat16),
                                     format=_ilv)
            for hh, av in ((2 * p, a_lo), (2 * p + 1, a_hi)):
                dv = plsc.load_gather(advl, [jnp.full((16,), hh, jnp.int32),
                                             d16])
                e = av + dv
                e = jnp.maximum(e, 0.2 * e)
                sv = jnp.exp(e)
                plsc.addupdate_scatter(acc, [jnp.full((16,), D + hh,
                                                      jnp.int32), d16], sv)
                for cw in range(HD // 2):
                    wv = plsc.load_gather(
                        hv[b], [iota16,
                                jnp.full((16,), hh * 8 + cw, jnp.int32)])
                    c_lo, c_hi = plsc.unpack(plsc.bitcast(wv, jnp.bfloat16),
                                             format=_ilv)
                    base = hh * HD + 2 * cw
                    plsc.addupdate_scatter(
                        acc, [jnp.full((16,), base, jnp.int32), d16],
                        c_lo * sv)
                    plsc.addupdate_scatter(
                        acc, [jnp.full((16,), base + 1, jnp.int32), d16],
                        c_hi * sv)

    issue(0, 0)
    issue(1, 1)

    def group_pair(i, carry):
        for b in (0, 1):
            g = 2 * i + b
            wait_group(b)
            accumulate(b)

            @pl.when(g + 2 < ng)
            def _():
                issue(g + 2, b)
        return carry

    lax.fori_loop(0, ng // 2, group_pair, 0)

    pltpu.sync_copy(acc, pm_hbm.at[w])


def _edge_call(hb_pk, asb_pk, adP, pkin):
    mesh = plsc.VectorSubcoreMesh(core_axis_name="c", subcore_axis_name="s")
    fn = pl.kernel(
        _edge_body,
        out_type=jax.ShapeDtypeStruct((NW, ACCW, R), jnp.float32),
        mesh=mesh,
        scratch_types=[
            pltpu.VMEM((ACCW, R), jnp.float32),
            pltpu.VMEM((PKCAP,), jnp.int32),
            pltpu.VMEM((N_PAD * HEADS // 2,), jnp.int32),
            pltpu.VMEM((16, R), jnp.float32),
            pltpu.VMEM((ECHUNK,), jnp.int32),
            pltpu.VMEM((ECHUNK,), jnp.int32),
            pltpu.VMEM((16, D // 2 + 8), jnp.int32),
            pltpu.VMEM((16, D // 2 + 8), jnp.int32),
            pltpu.VMEM((16,), jnp.int32),
            pltpu.VMEM((16,), jnp.int32),
            pltpu.VMEM((16,), jnp.int32),
            pltpu.VMEM((16,), jnp.int32),
            pltpu.SemaphoreType.DMA,
            pltpu.SemaphoreType.DMA,
            pltpu.SemaphoreType.DMA,
            pltpu.SemaphoreType.DMA,
        ],
        compiler_params=pltpu.CompilerParams(
            needs_layout_passes=False, use_tc_tiling_on_sc=False),
    )
    return fn(hb_pk, asb_pk, adP, pkin)


def _comb_body(pm_ref, h_ref, as_ref, ad_ref, b_ref, o_ref):
    pmb = pm_ref[0]
    msg = pmb[:D, :]
    den8 = pmb[D:D + HEADS, :]
    e = as_ref[0][:HEADS, :] + ad_ref[0][:HEADS, :]
    e = jnp.maximum(e, 0.2 * e)
    sself = jnp.exp(e)
    den = den8 + sself
    rows = lax.broadcasted_iota(jnp.int32, (D, HEADS), 0)
    cols = lax.broadcasted_iota(jnp.int32, (D, HEADS), 1)
    expand = (rows // HD == cols).astype(jnp.float32)
    den128 = jax.lax.dot(expand, den, precision=_HIGH)
    s128 = jax.lax.dot(expand, sself, precision=_HIGH)
    out = (msg + h_ref[0] * s128) / den128 + b_ref[...]
    o_ref[0] = jnp.maximum(out, 0.0)


def _combine(pm, hPT, asPT, adPT, biasT):
    return pl.pallas_call(
        _comb_body,
        grid=(NW,),
        in_specs=[
            pl.BlockSpec((1, ACCW, R), lambda t: (t, 0, 0)),
            pl.BlockSpec((1, D, R), lambda t: (t, 0, 0)),
            pl.BlockSpec((1, 16, R), lambda t: (t, 0, 0)),
            pl.BlockSpec((1, 16, R), lambda t: (t, 0, 0)),
            pl.BlockSpec((D, 1), lambda t: (0, 0)),
        ],
        out_specs=pl.BlockSpec((1, D, R), lambda t: (t, 0, 0)),
        out_shape=jax.ShapeDtypeStruct((NW, D, R), jnp.float32),
    )(pm, hPT, asPT, adPT, biasT)


def kernel(x, edge_index, W, att_src, att_dst, bias):
    src_lin = edge_index[0].astype(jnp.int32)
    dst_lin = edge_index[1].astype(jnp.int32)

    eye = jnp.eye(HEADS, dtype=jnp.float32)
    a_s = att_src.reshape(HEADS, HD)
    a_d = att_dst.reshape(HEADS, HD)
    AsM = (a_s[:, :, None] * eye[:, None, :]).reshape(D, HEADS)
    AdM = (a_d[:, :, None] * eye[:, None, :]).reshape(D, HEADS)
    pad = jnp.zeros((D, 16 - HEADS), jnp.float32)
    AsM = jnp.concatenate([AsM, pad], axis=1)
    AdM = jnp.concatenate([AdM, pad], axis=1)

    h, as16, ad16 = _dense_pre(x, W, AsM, AdM)

    hpad = jnp.pad(h, ((0, N_PAD - N_NODES), (0, 0)))
    aspad = jnp.pad(as16, ((0, N_PAD - N_NODES), (0, 0)))
    adpad = jnp.pad(ad16, ((0, N_PAD - N_NODES), (0, 0)))
    hPT = hpad.reshape(R, NW, D).transpose(1, 2, 0)
    asPT = aspad.reshape(R, NW, 16).transpose(1, 2, 0)
    adPT = adpad.reshape(R, NW, 16).transpose(1, 2, 0)

    hb_pk = lax.bitcast_convert_type(
        hpad.astype(jnp.bfloat16).reshape(N_PAD, D // 2, 2), jnp.int32)
    hb_pk = jnp.pad(hb_pk, ((0, 0), (0, 8)))
    asb_pk = lax.bitcast_convert_type(
        aspad[:, :HEADS].astype(jnp.bfloat16).reshape(N_PAD, HEADS // 2, 2),
        jnp.int32).reshape(N_PAD * HEADS // 2)
    pkin = src_lin | (dst_lin << 14)

    pm = _edge_call(hb_pk, asb_pk, adPT, pkin)

    biasT = bias.reshape(D, 1)
    outPT = _combine(pm, hPT, asPT, adPT, biasT)
    return outPT.transpose(2, 0, 1).reshape(N_PAD, D)[:N_NODES]


# speedup vs baseline: 1.7881x; 1.0081x over previous
import jax
import jax.numpy as jnp
from jax import lax
from jax.experimental import pallas as pl
from jax.experimental.pallas import tpu as pltpu
from jax.experimental.pallas import tpu_sc as plsc

N_NODES = 10000
N_PAD = 10240
D = 128
HEADS = 8
HD = 16
N_EDGES = 320000

NC = 2
NS = 16
NW = NC * NS
R = N_PAD // NW
ACCW = 144
ECHUNK = 4000
N_ECHUNKS = N_EDGES // ECHUNK
PKCAP = 12032
PADPK = (R - 1) << 14

_HIGH = jax.lax.Precision.HIGHEST


def _pre_body(x_ref, w_ref, am_ref, ad_ref, h_ref, as_ref, adr_ref):
    h = jax.lax.dot(x_ref[...], w_ref[...], precision=_HIGH)
    h_ref[...] = h
    as_ref[...] = jax.lax.dot(h, am_ref[...], precision=_HIGH)
    adr_ref[...] = jax.lax.dot(h, ad_ref[...], precision=_HIGH)


def _dense_pre(x, W, AsM, AdM):
    blk = 1000
    grid = N_NODES // blk
    return pl.pallas_call(
        _pre_body,
        grid=(grid,),
        in_specs=[
            pl.BlockSpec((blk, D), lambda i: (i, 0)),
            pl.BlockSpec((D, D), lambda i: (0, 0)),
            pl.BlockSpec((D, 16), lambda i: (0, 0)),
            pl.BlockSpec((D, 16), lambda i: (0, 0)),
        ],
        out_specs=[
            pl.BlockSpec((blk, D), lambda i: (i, 0)),
            pl.BlockSpec((blk, 16), lambda i: (i, 0)),
            pl.BlockSpec((blk, 16), lambda i: (i, 0)),
        ],
        out_shape=[
            jax.ShapeDtypeStruct((N_NODES, D), jnp.float32),
            jax.ShapeDtypeStruct((N_NODES, 16), jnp.float32),
            jax.ShapeDtypeStruct((N_NODES, 16), jnp.float32),
        ],
    )(x, W, AsM, AdM)


def _edge_body(hb_hbm, asb_hbm, adp_hbm, pk_hbm,
               pm_hbm,
               acc, pkbuf, asbv, advl,
               ebf0, ebf1, hv0, hv1,
               sidx0, sidx1, didx0, didx1,
               st0, st1, sg0, sg1):
    ebf = (ebf0, ebf1)
    hv = (hv0, hv1)
    sidx = (sidx0, sidx1)
    didx = (didx0, didx1)
    st = (st0, st1)
    sg = (sg0, sg1)

    c = lax.axis_index("c")
    s = lax.axis_index("s")
    w = c * NS + s

    zeros16 = jnp.zeros((16,), jnp.float32)
    iota16 = lax.iota(jnp.int32, 16)
    _ilv = plsc.PackFormat.INTERLEAVED

    def zfill(i, carry):
        for k in range(R // 16):
            acc[i, pl.ds(k * 16, 16)] = zeros16
        return carry

    lax.fori_loop(0, ACCW, zfill, 0)

    pltpu.async_copy(asb_hbm, asbv, sg0)
    pltpu.async_copy(adp_hbm.at[w], advl, sg1)

    def stage(ci, b):
        pltpu.async_copy(pk_hbm.at[pl.ds(ci * ECHUNK, ECHUNK)], ebf[b],
                         st[b])

    def wait_stage(ci, b):
        pltpu.make_async_copy(pk_hbm.at[pl.ds(ci * ECHUNK, ECHUNK)],
                              ebf[b], st[b]).wait()

    stage(0, 0)
    stage(1, 1)

    def scan_pair(i, o):
        for b in (0, 1):
            ci = 2 * i + b
            wait_stage(ci, b)

            def scan16(j, oo):
                pkin = ebf[b][pl.ds(j * 16, 16)]
                d16 = pkin >> 14
                m = (d16 & 31) == w
                entry = (pkin & 16383) | ((pkin >> 19) << 14)
                plsc.store_compressed(pkbuf.at[pl.ds(oo, 16)], entry,
                                      mask=m)
                cnt = plsc.all_reduce_population_count(m)
                return oo + cnt[0]

            o = lax.fori_loop(0, ECHUNK // 16, scan16, o)

            @pl.when(ci + 2 < N_ECHUNKS)
            def _():
                stage(ci + 2, b)
        return o

    o = lax.fori_loop(0, N_ECHUNKS // 2, scan_pair, jnp.int32(0))

    padvec = jnp.full((16,), PADPK, jnp.int32)
    pkbuf[pl.ds(o, 16)] = padvec
    pkbuf[pl.ds(o + 16, 16)] = padvec
    ng = ((o + 31) // 32) * 2

    pltpu.make_async_copy(asb_hbm, asbv, sg0).wait()
    pltpu.make_async_copy(adp_hbm.at[w], advl, sg1).wait()

    def issue(g, b):
        pk = pkbuf[pl.ds(g * 16, 16)]
        sidx[b][...] = pk & 16383
        didx[b][...] = pk >> 14
        pltpu.async_copy(hb_hbm.at[sidx[b]], hv[b], sg[b])

    def wait_group(b):
        pltpu.make_async_copy(hb_hbm.at[sidx[b]], hv[b], sg[b]).wait()

    def accumulate(b):
        d16 = didx[b][...]
        s16v = sidx[b][...]
        for p in range(HEADS // 2):
            was = plsc.load_gather(asbv, [s16v * 4 + p])
            a_lo, a_hi = plsc.unpack(plsc.bitcast(was, jnp.bfloat16),
                                     format=_ilv)
            for hh, av in ((2 * p, a_lo), (2 * p + 1, a_hi)):
                dv = plsc.load_gather(advl, [jnp.full((16,), hh, jnp.int32),
                                             d16])
                e = av + dv
                e = jnp.maximum(e, 0.2 * e)
                sv = jnp.exp(e)
                plsc.addupdate_scatter(acc, [jnp.full((16,), D + hh,
                                                      jnp.int32), d16], sv)
                for cw in range(HD // 2):
                    wv = plsc.load_gather(
                        hv[b], [iota16,
                                jnp.full((16,), hh * 8 + cw, jnp.int32)])
                    c_lo, c_hi = plsc.unpack(plsc.bitcast(wv, jnp.bfloat16),
                                             format=_ilv)
                    base = hh * HD + 2 * cw
                    plsc.addupdate_scatter(
                        acc, [jnp.full((16,), base, jnp.int32), d16],
                        c_lo * sv)
                    plsc.addupdate_scatter(
                        acc, [jnp.full((16,), base + 1, jnp.int32), d16],
                        c_hi * sv)

    issue(0, 0)
    issue(1, 1)

    def group_pair(i, carry):
        for b in (0, 1):
            g = 2 * i + b
            wait_group(b)
            accumulate(b)

            @pl.when(g + 2 < ng)
            def _():
                issue(g + 2, b)
        return carry

    lax.fori_loop(0, ng // 2, group_pair, 0)

    pltpu.sync_copy(acc, pm_hbm.at[w])


def _edge_call(hb_pk, asb_pk, adP, pkin):
    mesh = plsc.VectorSubcoreMesh(core_axis_name="c", subcore_axis_name="s")
    fn = pl.kernel(
        _edge_body,
        out_type=jax.ShapeDtypeStruct((NW, ACCW, R), jnp.float32),
        mesh=mesh,
        scratch_types=[
            pltpu.VMEM((ACCW, R), jnp.float32),
            pltpu.VMEM((PKCAP,), jnp.int32),
            pltpu.VMEM((N_PAD * HEADS // 2,), jnp.int32),
            pltpu.VMEM((16, R), jnp.float32),
            pltpu.VMEM((ECHUNK,), jnp.int32),
            pltpu.VMEM((ECHUNK,), jnp.int32),
            pltpu.VMEM((16, D // 2 + 8), jnp.int32),
            pltpu.VMEM((16, D // 2 + 8), jnp.int32),
            pltpu.VMEM((16,), jnp.int32),
            pltpu.VMEM((16,), jnp.int32),
            pltpu.VMEM((16,), jnp.int32),
            pltpu.VMEM((16,), jnp.int32),
            pltpu.SemaphoreType.DMA,
            pltpu.SemaphoreType.DMA,
            pltpu.SemaphoreType.DMA,
            pltpu.SemaphoreType.DMA,
        ],
        compiler_params=pltpu.CompilerParams(
            needs_layout_passes=False, use_tc_tiling_on_sc=False),
    )
    return fn(hb_pk, asb_pk, adP, pkin)


def _comb_body(pm_ref, h_ref, as_ref, ad_ref, b_ref, o_ref):
    pmb = pm_ref[0]
    msg = pmb[:D, :]
    den8 = pmb[D:D + HEADS, :]
    e = as_ref[0][:HEADS, :] + ad_ref[0][:HEADS, :]
    e = jnp.maximum(e, 0.2 * e)
    sself = jnp.exp(e)
    den = den8 + sself
    rows = lax.broadcasted_iota(jnp.int32, (D, HEADS), 0)
    cols = lax.broadcasted_iota(jnp.int32, (D, HEADS), 1)
    expand = (rows // HD == cols).astype(jnp.float32)
    den128 = jax.lax.dot(expand, den, precision=_HIGH)
    s128 = jax.lax.dot(expand, sself, precision=_HIGH)
    out = (msg + h_ref[0] * s128) / den128 + b_ref[...]
    o_ref[0] = jnp.maximum(out, 0.0)


def _combine(pm, hPT, asPT, adPT, biasT):
    return pl.pallas_call(
        _comb_body,
        grid=(NW,),
        in_specs=[
            pl.BlockSpec((1, ACCW, R), lambda t: (t, 0, 0)),
            pl.BlockSpec((1, D, R), lambda t: (t, 0, 0)),
            pl.BlockSpec((1, 16, R), lambda t: (t, 0, 0)),
            pl.BlockSpec((1, 16, R), lambda t: (t, 0, 0)),
            pl.BlockSpec((D, 1), lambda t: (0, 0)),
        ],
        out_specs=pl.BlockSpec((1, D, R), lambda t: (t, 0, 0)),
        out_shape=jax.ShapeDtypeStruct((NW, D, R), jnp.float32),
    )(pm, hPT, asPT, adPT, biasT)


def kernel(x, edge_index, W, att_src, att_dst, bias):
    src_lin = edge_index[0].astype(jnp.int32)
    dst_lin = edge_index[1].astype(jnp.int32)

    eye = jnp.eye(HEADS, dtype=jnp.float32)
    a_s = att_src.reshape(HEADS, HD)
    a_d = att_dst.reshape(HEADS, HD)
    AsM = (a_s[:, :, None] * eye[:, None, :]).reshape(D, HEADS)
    AdM = (a_d[:, :, None] * eye[:, None, :]).reshape(D, HEADS)
    pad = jnp.zeros((D, 16 - HEADS), jnp.float32)
    AsM = jnp.concatenate([AsM, pad], axis=1)
    AdM = jnp.concatenate([AdM, pad], axis=1)

    h, as16, ad16 = _dense_pre(x, W, AsM, AdM)

    hpad = jnp.pad(h, ((0, N_PAD - N_NODES), (0, 0)))
    aspad = jnp.pad(as16, ((0, N_PAD - N_NODES), (0, 0)))
    adpad = jnp.pad(ad16, ((0, N_PAD - N_NODES), (0, 0)))
    hPT = hpad.reshape(R, NW, D).transpose(1, 2, 0)
    asPT = aspad.reshape(R, NW, 16).transpose(1, 2, 0)
    adPT = adpad.reshape(R, NW, 16).transpose(1, 2, 0)

    hb_pk = lax.bitcast_convert_type(
        hpad.astype(jnp.bfloat16).reshape(N_PAD, D // 2, 2), jnp.int32)
    hb_pk = jnp.pad(hb_pk, ((0, 0), (0, 8)))
    asb_pk = lax.bitcast_convert_type(
        aspad[:, :HEADS].astype(jnp.bfloat16).reshape(N_PAD, HEADS // 2, 2),
        jnp.int32).reshape(N_PAD * HEADS // 2)
    pkin = src_lin | (dst_lin << 14)

    pm = _edge_call(hb_pk, asb_pk, adPT, pkin)

    biasT = bias.reshape(D, 1)
    outPT = _combine(pm, hPT, asPT, adPT, biasT)
    return outPT.transpose(2, 0, 1).reshape(N_PAD, D)[:N_NODES]
